# Initial kernel scaffold; baseline (speedup 1.0000x reference)
import jax
import jax.numpy as jnp
from jax.experimental import pallas as pl


def kernel(graph, data, W1, b1, W2, b2):
    raise NotImplementedError("write your pallas kernel here")


# speedup vs baseline: 19.5336x; 19.5336x over previous
import functools

import jax
import jax.numpy as jnp
from jax import lax
from jax.experimental import pallas as pl
from jax.experimental.pallas import tpu as pltpu
from jax.experimental.pallas import tpu_sc as plsc

N = 10000
E = 320000
IN_FEATS = 128
HID = 16
NUM_CLASSES = 40
CPAD = 48

NTILES = 32
CH = 128
NP = 10112
SLC = NP // 16
EPT = 10112
NCH = EPT // CH
EP = EPT * NTILES


def _mesh():
    return plsc.VectorSubcoreMesh(core_axis_name="c", subcore_axis_name="s")


def _sc_degree(dstp, zeros1, ones1):
    @functools.partial(
        pl.kernel,
        out_type=jax.ShapeDtypeStruct((2 * NP,), jnp.float32),
        mesh=_mesh(),
        scratch_types=[
            pltpu.VMEM((CH,), jnp.int32),
            pltpu.VMEM((CH,), jnp.float32),
            pltpu.VMEM((SLC,), jnp.float32),
            pltpu.VMEM_SHARED((NP,), jnp.float32),
        ],
    )
    def k(dst_hbm, z_hbm, ones_hbm, out_hbm, didx, ones_v, slab, accum):
        c = lax.axis_index("c")
        s = lax.axis_index("s")
        pltpu.sync_copy(ones_hbm, ones_v)
        pltpu.sync_copy(z_hbm.at[pl.ds(s * SLC, SLC)], slab)
        pltpu.sync_copy(slab, accum.at[pl.ds(s * SLC, SLC)])
        plsc.subcore_barrier()
        base = (c * 16 + s) * EPT

        def body(g, carry):
            pltpu.sync_copy(dst_hbm.at[pl.ds(base + g * CH, CH)], didx)
            pltpu.sync_copy(ones_v, accum.at[didx], add=True)
            return carry

        lax.fori_loop(0, NCH, body, 0)
        plsc.subcore_barrier()
        pltpu.sync_copy(accum.at[pl.ds(s * SLC, SLC)], slab)
        pltpu.sync_copy(slab, out_hbm.at[pl.ds(c * NP + s * SLC, SLC)])

    return k(dstp, zeros1, ones1).reshape(2, NP)


def _sc_agg(srcp, dstp, y, zeros2, d):
    @functools.partial(
        pl.kernel,
        out_type=jax.ShapeDtypeStruct((2, NP, d), jnp.float32),
        mesh=_mesh(),
        scratch_types=[
            pltpu.VMEM((CH,), jnp.int32),
            pltpu.VMEM((CH,), jnp.int32),
            pltpu.VMEM((CH, d), jnp.float32),
            pltpu.VMEM((SLC, d), jnp.float32),
            pltpu.VMEM_SHARED((NP, d), jnp.float32),
            pltpu.SemaphoreType.DMA,
        ],
        compiler_params=pltpu.CompilerParams(use_tc_tiling_on_sc=False),
    )
    def k(src_hbm, dst_hbm, y_hbm, z_hbm, out_hbm, sidx, didx, rows, slab, accum, sem):
        c = lax.axis_index("c")
        s = lax.axis_index("s")
        pltpu.sync_copy(z_hbm.at[pl.ds(s * SLC, SLC)], slab)
        pltpu.sync_copy(slab, accum.at[pl.ds(s * SLC, SLC)])
        plsc.subcore_barrier()
        base = (c * 16 + s) * EPT

        def body(g, carry):
            e0 = base + g * CH
            pltpu.sync_copy(src_hbm.at[pl.ds(e0, CH)], sidx)
            pltpu.sync_copy(dst_hbm.at[pl.ds(e0, CH)], didx)
            pltpu.async_copy(y_hbm.at[sidx], rows, sem).wait()
            pltpu.sync_copy(rows, accum.at[didx], add=True)
            return carry

        lax.fori_loop(0, NCH, body, 0)
        plsc.subcore_barrier()
        pltpu.sync_copy(accum.at[pl.ds(s * SLC, SLC)], slab)
        pltpu.sync_copy(slab, out_hbm.at[c, pl.ds(s * SLC, SLC)])

    return k(srcp, dstp, y, zeros2)


def _dinv(dp_ref):
    deg = dp_ref[0:1, :] + dp_ref[1:2, :] + 1.0
    return lax.rsqrt(deg)


def _tc1_body(x_ref, w_ref, dp_ref, y_ref):
    dinv = _dinv(dp_ref)
    xw = jnp.dot(x_ref[...], w_ref[...], preferred_element_type=jnp.float32)
    row = lax.broadcasted_iota(jnp.int32, (NP, HID), 0)
    y_ref[...] = jnp.where(row < N, xw * dinv.reshape(NP, 1), 0.0)


def _tc2_body(s1_ref, y1_ref, b1_ref, dp_ref, w2_ref, y2_ref):
    dinv = _dinv(dp_ref).reshape(NP, 1)
    t = s1_ref[0] + s1_ref[1] + y1_ref[...]
    h = jnp.maximum(t * dinv + b1_ref[...], 0.0)
    y2 = jnp.dot(h, w2_ref[...], preferred_element_type=jnp.float32) * dinv
    row = lax.broadcasted_iota(jnp.int32, (NP, CPAD), 0)
    y2_ref[...] = jnp.where(row < N, y2, 0.0)


def _tc3_body(s2_ref, y2_ref, b2_ref, dp_ref, o_ref):
    dinv = _dinv(dp_ref).reshape(NP, 1)
    logits = (s2_ref[0] + s2_ref[1] + y2_ref[...]) * dinv + b2_ref[...]
    col = lax.broadcasted_iota(jnp.int32, (NP, CPAD), 1)
    valid = col < NUM_CLASSES
    lm = jnp.where(valid, logits, -1e30)
    mx = jnp.max(lm, axis=1, keepdims=True)
    se = jnp.sum(jnp.where(valid, jnp.exp(logits - mx), 0.0), axis=1, keepdims=True)
    o_ref[...] = logits - mx - jnp.log(se)


def kernel(graph, data, W1, b1, W2, b2):
    f32 = jnp.float32
    src = graph[0]
    dst = graph[1]
    pad = EP - E
    srcp = jnp.concatenate([src, jnp.full((pad,), N, jnp.int32)])
    dstp = jnp.concatenate([dst, jnp.full((pad,), N, jnp.int32)])
    datap = jnp.pad(data.astype(f32), ((0, NP - N), (0, 0)))
    w2p = jnp.pad(W2.astype(f32), ((0, 0), (0, CPAD - NUM_CLASSES)))
    b1r = b1.astype(f32).reshape(1, HID)
    b2r = jnp.pad(b2.astype(f32), (0, CPAD - NUM_CLASSES)).reshape(1, CPAD)

    zeros1 = jnp.zeros((NP,), f32)
    ones1 = jnp.ones((CH,), f32)
    zeros_h = jnp.zeros((NP, HID), f32)
    zeros_c = jnp.zeros((NP, CPAD), f32)

    dp = _sc_degree(dstp, zeros1, ones1)

    y1 = pl.pallas_call(
        _tc1_body,
        out_shape=jax.ShapeDtypeStruct((NP, HID), f32),
    )(datap, W1.astype(f32), dp)

    s1 = _sc_agg(srcp, dstp, y1, zeros_h, HID)

    y2 = pl.pallas_call(
        _tc2_body,
        out_shape=jax.ShapeDtypeStruct((NP, CPAD), f32),
    )(s1, y1, b1r, dp, w2p)

    s2 = _sc_agg(srcp, dstp, y2, zeros_c, CPAD)

    out = pl.pallas_call(
        _tc3_body,
        out_shape=jax.ShapeDtypeStruct((NP, CPAD), f32),
    )(s2, y2, b2r, dp)

    return out[:N, :NUM_CLASSES]


# speedup vs baseline: 26.9793x; 1.3812x over previous
import functools

import jax
import jax.numpy as jnp
from jax import lax
from jax.experimental import pallas as pl
from jax.experimental.pallas import tpu as pltpu
from jax.experimental.pallas import tpu_sc as plsc

N = 10000
E = 320000
IN_FEATS = 128
HID = 16
NUM_CLASSES = 40
CPAD = 48

NTILES = 32
CH = 128
NP = 10112
SLC = NP // 16
NCH = 80
EPT = NCH * CH
EP = EPT * NTILES
NBUF = 4


def _mesh():
    return plsc.VectorSubcoreMesh(core_axis_name="c", subcore_axis_name="s")


def _sc_degree(dst2d, zeros1, ones1):
    @functools.partial(
        pl.kernel,
        out_type=jax.ShapeDtypeStruct((2 * NP,), jnp.float32),
        mesh=_mesh(),
        scratch_types=[
            pltpu.VMEM((NCH, CH), jnp.int32),
            pltpu.VMEM((CH,), jnp.float32),
            pltpu.VMEM((SLC,), jnp.float32),
            pltpu.VMEM_SHARED((NP,), jnp.float32),
            pltpu.SemaphoreType.DMA,
        ],
        compiler_params=pltpu.CompilerParams(use_tc_tiling_on_sc=False),
    )
    def k(dst_hbm, z_hbm, ones_hbm, out_hbm, didx, ones_v, slab, accum, sem):
        c = lax.axis_index("c")
        s = lax.axis_index("s")
        wid = c * 16 + s
        pltpu.sync_copy(ones_hbm, ones_v)
        pltpu.sync_copy(dst_hbm.at[pl.ds(wid * NCH, NCH)], didx)
        pltpu.sync_copy(z_hbm.at[pl.ds(s * SLC, SLC)], slab)
        pltpu.sync_copy(slab, accum.at[pl.ds(s * SLC, SLC)])
        plsc.subcore_barrier()

        def body(g, carry):
            descs = [
                pltpu.async_copy(ones_v, accum.at[didx.at[g * 8 + j]], sem, add=True)
                for j in range(8)
            ]
            for d in descs:
                d.wait()
            return carry

        lax.fori_loop(0, NCH // 8, body, 0)
        plsc.subcore_barrier()
        pltpu.sync_copy(accum.at[pl.ds(s * SLC, SLC)], slab)
        pltpu.sync_copy(slab, out_hbm.at[pl.ds(c * NP + s * SLC, SLC)])

    return k(dst2d, zeros1, ones1).reshape(2, NP)


def _sc_agg(srcp, dst2d, y, zeros2, d):
    @functools.partial(
        pl.kernel,
        out_type=jax.ShapeDtypeStruct((2, NP, d), jnp.float32),
        mesh=_mesh(),
        scratch_types=[
            pltpu.VMEM((EPT,), jnp.int32),
            pltpu.VMEM((NCH, CH), jnp.int32),
            [pltpu.VMEM((CH, d), jnp.float32) for _ in range(NBUF)],
            pltpu.VMEM((SLC, d), jnp.float32),
            pltpu.VMEM_SHARED((NP, d), jnp.float32),
            [pltpu.SemaphoreType.DMA for _ in range(NBUF)],
        ],
        compiler_params=pltpu.CompilerParams(use_tc_tiling_on_sc=False),
    )
    def k(src_hbm, dst_hbm, y_hbm, z_hbm, out_hbm, sidx, didx, rows, slab, accum, gsem):
        c = lax.axis_index("c")
        s = lax.axis_index("s")
        wid = c * 16 + s
        pltpu.sync_copy(src_hbm.at[pl.ds(wid * EPT, EPT)], sidx)
        pltpu.sync_copy(dst_hbm.at[pl.ds(wid * NCH, NCH)], didx)
        pltpu.sync_copy(z_hbm.at[pl.ds(s * SLC, SLC)], slab)
        pltpu.sync_copy(slab, accum.at[pl.ds(s * SLC, SLC)])
        plsc.subcore_barrier()

        def gather(ch, j):
            return pltpu.async_copy(
                y_hbm.at[sidx.at[pl.ds(ch * CH, CH)]], rows[j], gsem[j]
            )

        def scatter(ch, j):
            pltpu.sync_copy(rows[j], accum.at[didx.at[ch]], add=True)

        def gsem_wait(j):
            pltpu.make_async_copy(
                y_hbm.at[sidx.at[pl.ds(0, CH)]], rows[j], gsem[j]
            ).wait()

        for j in range(NBUF):
            gather(j, j)

        def body(g, carry):
            c0 = g * NBUF
            for j in range(NBUF):
                gsem_wait(j)
                scatter(c0 + j, j)
                gather(c0 + NBUF + j, j)
            return carry

        lax.fori_loop(0, NCH // NBUF - 1, body, 0)
        for j in range(NBUF):
            gsem_wait(j)
            scatter(NCH - NBUF + j, j)
        plsc.subcore_barrier()
        pltpu.sync_copy(accum.at[pl.ds(s * SLC, SLC)], slab)
        pltpu.sync_copy(slab, out_hbm.at[c, pl.ds(s * SLC, SLC)])

    return k(srcp, dst2d, y, zeros2)


def _dinv(dp_ref):
    deg = dp_ref[0:1, :] + dp_ref[1:2, :] + 1.0
    return lax.rsqrt(deg)


def _tc1_body(x_ref, w_ref, dp_ref, y_ref):
    dinv = _dinv(dp_ref)
    xw = jnp.dot(x_ref[...], w_ref[...], preferred_element_type=jnp.float32)
    row = lax.broadcasted_iota(jnp.int32, (NP, HID), 0)
    y_ref[...] = jnp.where(row < N, xw * dinv.reshape(NP, 1), 0.0)


def _tc2_body(s1_ref, y1_ref, b1_ref, dp_ref, w2_ref, y2_ref):
    dinv = _dinv(dp_ref).reshape(NP, 1)
    t = s1_ref[0] + s1_ref[1] + y1_ref[...]
    h = jnp.maximum(t * dinv + b1_ref[...], 0.0)
    y2 = jnp.dot(h, w2_ref[...], preferred_element_type=jnp.float32) * dinv
    row = lax.broadcasted_iota(jnp.int32, (NP, CPAD), 0)
    y2_ref[...] = jnp.where(row < N, y2, 0.0)


def _tc3_body(s2_ref, y2_ref, b2_ref, dp_ref, o_ref):
    dinv = _dinv(dp_ref).reshape(NP, 1)
    logits = (s2_ref[0] + s2_ref[1] + y2_ref[...]) * dinv + b2_ref[...]
    col = lax.broadcasted_iota(jnp.int32, (NP, CPAD), 1)
    valid = col < NUM_CLASSES
    lm = jnp.where(valid, logits, -1e30)
    mx = jnp.max(lm, axis=1, keepdims=True)
    se = jnp.sum(jnp.where(valid, jnp.exp(logits - mx), 0.0), axis=1, keepdims=True)
    o_ref[...] = logits - mx - jnp.log(se)


def kernel(graph, data, W1, b1, W2, b2):
    f32 = jnp.float32
    src = graph[0]
    dst = graph[1]
    pad = EP - E
    srcp = jnp.concatenate([src, jnp.full((pad,), N, jnp.int32)])
    dstp = jnp.concatenate([dst, jnp.full((pad,), N, jnp.int32)])
    dst2d = dstp.reshape(NTILES * NCH, CH)
    datap = jnp.pad(data.astype(f32), ((0, NP - N), (0, 0)))
    w2p = jnp.pad(W2.astype(f32), ((0, 0), (0, CPAD - NUM_CLASSES)))
    b1r = b1.astype(f32).reshape(1, HID)
    b2r = jnp.pad(b2.astype(f32), (0, CPAD - NUM_CLASSES)).reshape(1, CPAD)

    zeros1 = jnp.zeros((NP,), f32)
    ones1 = jnp.ones((CH,), f32)
    zeros_h = jnp.zeros((NP, HID), f32)
    zeros_c = jnp.zeros((NP, CPAD), f32)

    dp = _sc_degree(dst2d, zeros1, ones1)

    y1 = pl.pallas_call(
        _tc1_body,
        out_shape=jax.ShapeDtypeStruct((NP, HID), f32),
    )(datap, W1.astype(f32), dp)

    s1 = _sc_agg(srcp, dst2d, y1, zeros_h, HID)

    y2 = pl.pallas_call(
        _tc2_body,
        out_shape=jax.ShapeDtypeStruct((NP, CPAD), f32),
    )(s1, y1, b1r, dp, w2p)

    s2 = _sc_agg(srcp, dst2d, y2, zeros_c, CPAD)

    out = pl.pallas_call(
        _tc3_body,
        out_shape=jax.ShapeDtypeStruct((NP, CPAD), f32),
    )(s2, y2, b2r, dp)

    return out[:N, :NUM_CLASSES]
